# Initial kernel scaffold; baseline (speedup 1.0000x reference)
import jax
import jax.numpy as jnp
from jax.experimental import pallas as pl


def kernel(x_1, edge_index_1, x_1_batch, x_2, edge_index_2, x_2_batch, enc_W1, enc_b1, enc_W2, enc_b2, enc_W3, enc_b3, enc_W4, enc_b4, enc_W5, enc_b5, cls_W1, cls_b1, cls_W2, cls_b2, cls_W3, cls_b3, cls_W4, cls_b4, cls_W5, cls_b5):
    raise NotImplementedError("write your pallas kernel here")


# speedup vs baseline: 6.1399x; 6.1399x over previous
import functools

import jax
import jax.numpy as jnp
from jax import lax
from jax.experimental import pallas as pl
from jax.experimental.pallas import tpu as pltpu
from jax.experimental.pallas import tpu_sc as plsc

_N = 10000
_E = 320000
_D = 128
_G = 32
_NC = 2
_NS = 16
_NW = _NC * _NS
_EPW = _E // _NW
_CH = 80
_NSTEP = _EPW // _CH
_NP = 10240
_RPT = _NP // _NS
_WB = 80
_R = 2000
_F32 = jnp.float32


@functools.cache
def _sc_scatter_kernel():
    mesh = plsc.VectorSubcoreMesh(core_axis_name="c", subcore_axis_name="s")
    return functools.partial(
        pl.kernel, mesh=mesh,
        out_type=jax.ShapeDtypeStruct((_NC * _NP, _D), _F32),
        scratch_types=[
            pltpu.VMEM((1, _CH), jnp.int32),
            pltpu.VMEM((1, _CH), jnp.int32),
            pltpu.VMEM((_WB, _D), _F32),
            pltpu.VMEM_SHARED((_NP, _D), _F32),
            pltpu.SemaphoreType.DMA,
        ])(_sc_scatter_body)


def _sc_scatter_body(t_hbm, src_hbm, dst_hbm, zeros_hbm, out_hbm,
                     src_slot, dst_slot, rows_v, acc, sem):
    c = lax.axis_index("c")
    s = lax.axis_index("s")
    wid = c * _NS + s
    row0 = s * _RPT
    pltpu.sync_copy(zeros_hbm, rows_v)
    for k in range(_RPT // _WB):
        pltpu.sync_copy(rows_v, acc.at[pl.ds(row0 + k * _WB, _WB)])
    plsc.subcore_barrier()
    base = wid * _NSTEP

    def body(j, carry):
        pltpu.sync_copy(src_hbm.at[base + j], src_slot)
        pltpu.sync_copy(dst_hbm.at[base + j], dst_slot)
        pltpu.async_copy(t_hbm.at[src_slot.at[0]], rows_v, sem).wait()
        pltpu.sync_copy(rows_v, acc.at[dst_slot.at[0]], add=True)
        return carry

    lax.fori_loop(0, _NSTEP, body, 0)
    plsc.subcore_barrier()
    for k in range(_RPT // _WB):
        pltpu.sync_copy(acc.at[pl.ds(row0 + k * _WB, _WB)], rows_v)
        pltpu.sync_copy(rows_v, out_hbm.at[pl.ds(c * _NP + row0 + k * _WB,
                                                 _WB)])


def _first_body(x_ref, w_ref, degp_ref, t_ref, dinv_ref):
    deg = degp_ref[0][:, :1] + degp_ref[1][:, :1] + 1.0
    dinv = lax.rsqrt(deg)
    t_ref[...] = jnp.dot(x_ref[...], w_ref[...],
                         preferred_element_type=_F32) * dinv
    dinv_ref[...] = dinv


def _tc_first(x, w, degp):
    return pl.pallas_call(
        _first_body,
        grid=(_N // _R,),
        in_specs=[
            pl.BlockSpec((_R, _D), lambda i: (i, 0)),
            pl.BlockSpec((_D, _D), lambda i: (0, 0)),
            pl.BlockSpec((2, _R, _D), lambda i: (0, i, 0)),
        ],
        out_specs=[
            pl.BlockSpec((_R, _D), lambda i: (i, 0)),
            pl.BlockSpec((_R, 1), lambda i: (i, 0)),
        ],
        out_shape=[
            jax.ShapeDtypeStruct((_N, _D), _F32),
            jax.ShapeDtypeStruct((_N, 1), _F32),
        ],
    )(x, w, degp)


def _node_features(sp_ref, t_ref, dinv_ref, b_ref):
    s = sp_ref[0] + sp_ref[1] + t_ref[...]
    return jnp.maximum(dinv_ref[...] * s + b_ref[...], 0.0)


def _mid_body(sp_ref, t_ref, dinv_ref, b_ref, w_ref, tn_ref):
    x = _node_features(sp_ref, t_ref, dinv_ref, b_ref)
    tn_ref[...] = jnp.dot(x, w_ref[...],
                          preferred_element_type=_F32) * dinv_ref[...]


def _tc_mid(sp, t, dinv, b, w):
    return pl.pallas_call(
        _mid_body,
        grid=(_N // _R,),
        in_specs=[
            pl.BlockSpec((2, _R, _D), lambda i: (0, i, 0)),
            pl.BlockSpec((_R, _D), lambda i: (i, 0)),
            pl.BlockSpec((_R, 1), lambda i: (i, 0)),
            pl.BlockSpec((1, _D), lambda i: (0, 0)),
            pl.BlockSpec((_D, _D), lambda i: (0, 0)),
        ],
        out_specs=pl.BlockSpec((_R, _D), lambda i: (i, 0)),
        out_shape=jax.ShapeDtypeStruct((_N, _D), _F32),
    )(sp, t, dinv, b, w)


def _pool_body(sp_ref, t_ref, dinv_ref, b_ref, batch_ref, sums_ref, cnt_ref):
    i = pl.program_id(0)
    x = _node_features(sp_ref, t_ref, dinv_ref, b_ref)
    seg = lax.broadcasted_iota(jnp.int32, (_R, _G), 1)
    p = (batch_ref[...] == seg).astype(_F32)
    sums = lax.dot_general(p, x, (((0,), (0,)), ((), ())),
                           preferred_element_type=_F32)
    cnt = lax.dot_general(p, jnp.ones((_R, 1), _F32),
                          (((0,), (0,)), ((), ())),
                          preferred_element_type=_F32)

    @pl.when(i == 0)
    def _():
        sums_ref[...] = jnp.zeros_like(sums_ref)
        cnt_ref[...] = jnp.zeros_like(cnt_ref)

    sums_ref[...] += sums
    cnt_ref[...] += cnt


def _tc_pool(sp, t, dinv, b, batch):
    return pl.pallas_call(
        _pool_body,
        grid=(_N // _R,),
        in_specs=[
            pl.BlockSpec((2, _R, _D), lambda i: (0, i, 0)),
            pl.BlockSpec((_R, _D), lambda i: (i, 0)),
            pl.BlockSpec((_R, 1), lambda i: (i, 0)),
            pl.BlockSpec((1, _D), lambda i: (0, 0)),
            pl.BlockSpec((_R, 1), lambda i: (i, 0)),
        ],
        out_specs=[
            pl.BlockSpec((_G, _D), lambda i: (0, 0)),
            pl.BlockSpec((_G, 1), lambda i: (0, 0)),
        ],
        out_shape=[
            jax.ShapeDtypeStruct((_G, _D), _F32),
            jax.ShapeDtypeStruct((_G, 1), _F32),
        ],
    )(sp, t, dinv, b, batch)


def _mlp_body(s1_ref, c1_ref, s2_ref, c2_ref,
              w1_ref, b1_ref, w2_ref, b2_ref, w3_ref, b3_ref,
              w4_ref, b4_ref, w5_ref, b5_ref, out_ref):
    h1 = s1_ref[...] / jnp.maximum(c1_ref[...], 1.0)
    h2 = s2_ref[...] / jnp.maximum(c2_ref[...], 1.0)
    h = jnp.concatenate([h1, h2], axis=1)
    ws = [w1_ref, w2_ref, w3_ref, w4_ref, w5_ref]
    bs = [b1_ref, b2_ref, b3_ref, b4_ref, b5_ref]
    for li in range(5):
        h = jnp.dot(h, ws[li][...], preferred_element_type=_F32) + bs[li][...]
        if li < 4:
            h = jnp.maximum(h, 0.0)
    out_ref[...] = 1.0 / (1.0 + jnp.exp(-h))


def _tc_mlp(s1, c1, s2, c2, cls_params):
    flat = []
    for (w, b) in cls_params:
        flat.extend([w, b.reshape(1, -1)])
    return pl.pallas_call(
        _mlp_body,
        out_shape=jax.ShapeDtypeStruct((_G, 1), _F32),
    )(s1, c1, s2, c2, *flat)


def _encode(x, edge_index, batch, enc_params, zeros_row, ones_t):
    src3 = edge_index[0].astype(jnp.int32).reshape(_NW * _NSTEP, 1, _CH)
    dst3 = edge_index[1].astype(jnp.int32).reshape(_NW * _NSTEP, 1, _CH)
    degp = _sc_scatter_kernel()(ones_t, src3, dst3, zeros_row).reshape(
        _NC, _NP, _D)
    t, dinv = _tc_first(x, enc_params[0][0], degp)
    for li in range(1, 5):
        sp = _sc_scatter_kernel()(t, src3, dst3, zeros_row).reshape(
            _NC, _NP, _D)
        t = _tc_mid(sp, t, dinv, enc_params[li - 1][1].reshape(1, _D),
                    enc_params[li][0])
    sp = _sc_scatter_kernel()(t, src3, dst3, zeros_row).reshape(_NC, _NP, _D)
    batch2 = batch.astype(jnp.int32).reshape(_N, 1)
    return _tc_pool(sp, t, dinv, enc_params[4][1].reshape(1, _D), batch2)


def kernel(x_1, edge_index_1, x_1_batch, x_2, edge_index_2, x_2_batch,
           enc_W1, enc_b1, enc_W2, enc_b2, enc_W3, enc_b3, enc_W4, enc_b4,
           enc_W5, enc_b5, cls_W1, cls_b1, cls_W2, cls_b2, cls_W3, cls_b3,
           cls_W4, cls_b4, cls_W5, cls_b5):
    enc = [(enc_W1, enc_b1), (enc_W2, enc_b2), (enc_W3, enc_b3),
           (enc_W4, enc_b4), (enc_W5, enc_b5)]
    cls = [(cls_W1, cls_b1), (cls_W2, cls_b2), (cls_W3, cls_b3),
           (cls_W4, cls_b4), (cls_W5, cls_b5)]
    zeros_row = jnp.zeros((_WB, _D), _F32)
    ones_t = jnp.ones((_N, _D), _F32)
    s1, c1 = _encode(x_1, edge_index_1, x_1_batch, enc, zeros_row, ones_t)
    s2, c2 = _encode(x_2, edge_index_2, x_2_batch, enc, zeros_row, ones_t)
    return _tc_mlp(s1, c1, s2, c2, cls)


# speedup vs baseline: 17.2434x; 2.8084x over previous
import functools

import jax
import jax.numpy as jnp
from jax import lax
from jax.experimental import pallas as pl
from jax.experimental.pallas import tpu as pltpu
from jax.experimental.pallas import tpu_sc as plsc

_N = 10000
_E = 320000
_D = 128
_G = 32
_NC = 2
_NS = 16
_NW = _NC * _NS
_EPW = _E // _NW
_CH = 125
_GS = 4
_NG = 20
_NSTEP = _GS * _NG
_NP = 10240
_RPT = _NP // _NS
_WB = 64
_R = 2000
_F32 = jnp.float32


@functools.cache
def _sc_scatter_kernel():
    mesh = plsc.VectorSubcoreMesh(core_axis_name="c", subcore_axis_name="s")
    return functools.partial(
        pl.kernel, mesh=mesh,
        out_type=jax.ShapeDtypeStruct((_NC * _NP, _D), _F32),
        scratch_types=[
            pltpu.VMEM((_GS, _CH), jnp.int32),
            pltpu.VMEM((_GS, _CH), jnp.int32),
            pltpu.VMEM((_GS, _CH), jnp.int32),
            pltpu.VMEM((_GS, _CH), jnp.int32),
            pltpu.VMEM((_CH, _D), _F32),
            pltpu.VMEM((_CH, _D), _F32),
            pltpu.VMEM_SHARED((_NP, _D), _F32),
            pltpu.SemaphoreType.DMA,
            pltpu.SemaphoreType.DMA,
            pltpu.SemaphoreType.DMA,
            pltpu.SemaphoreType.DMA,
        ])(_sc_scatter_body)


def _sc_scatter_body(t_hbm, src_hbm, dst_hbm, zeros_hbm, out_hbm,
                     srcg0, dstg0, srcg1, dstg1, rows0, rows1, acc,
                     semi0, semi1, semg0, semg1):
    c = lax.axis_index("c")
    s = lax.axis_index("s")
    wid = c * _NS + s
    row0 = s * _RPT
    gbase = wid * _NG
    srcg, dstg = (srcg0, srcg1), (dstg0, dstg1)
    rows, semi, semg = (rows0, rows1), (semi0, semi1), (semg0, semg1)

    zbuf = rows0.at[pl.ds(0, _WB)]
    pltpu.sync_copy(zeros_hbm, zbuf)
    for k in range(_RPT // _WB):
        pltpu.sync_copy(zbuf, acc.at[pl.ds(row0 + k * _WB, _WB)])
    plsc.subcore_barrier()

    def stage(g_dyn, r):
        pltpu.async_copy(src_hbm.at[gbase + g_dyn], srcg[r], semi[r])
        pltpu.async_copy(dst_hbm.at[gbase + g_dyn], dstg[r], semi[r])

    def stage_wait(r):
        pltpu.make_async_copy(src_hbm.at[gbase], srcg[r], semi[r]).wait()
        pltpu.make_async_copy(dst_hbm.at[gbase], dstg[r], semi[r]).wait()

    def g_start(b):
        rg = (b // _GS) % 2
        pltpu.async_copy(t_hbm.at[srcg[rg].at[b % _GS]], rows[b % 2],
                         semg[b % 2])

    def g_wait(b):
        rg = (b // _GS) % 2
        pltpu.make_async_copy(t_hbm.at[srcg[rg].at[b % _GS]], rows[b % 2],
                              semg[b % 2]).wait()

    def emit(b, last=False):
        nxt = b + 1
        if not (last and b == 2 * _GS - 1):
            if nxt % _GS == 0:
                stage_wait((nxt // _GS) % 2)
            g_start(nxt)
        g_wait(b)
        rg = (b // _GS) % 2
        pltpu.sync_copy(rows[b % 2], acc.at[dstg[rg].at[b % _GS]], add=True)

    stage(0, 0)
    stage_wait(0)
    g_start(0)
    stage(1, 1)

    def body(i, carry):
        for b in range(_GS):
            emit(b)
        stage(2 * i + 2, 0)
        for b in range(_GS, 2 * _GS):
            emit(b)
        stage(2 * i + 3, 1)
        return carry

    lax.fori_loop(0, _NG // 2 - 1, body, 0)
    for b in range(2 * _GS):
        emit(b, last=True)

    plsc.subcore_barrier()
    for k in range(_RPT // _WB):
        pltpu.sync_copy(acc.at[pl.ds(row0 + k * _WB, _WB)], zbuf)
        pltpu.sync_copy(zbuf, out_hbm.at[pl.ds(c * _NP + row0 + k * _WB,
                                               _WB)])


def _first_body(x_ref, w_ref, degp_ref, t_ref, dinv_ref):
    deg = degp_ref[0][:, :1] + degp_ref[1][:, :1] + 1.0
    dinv = lax.rsqrt(deg)
    t_ref[...] = jnp.dot(x_ref[...], w_ref[...],
                         preferred_element_type=_F32) * dinv
    dinv_ref[...] = dinv


def _tc_first(x, w, degp):
    return pl.pallas_call(
        _first_body,
        grid=(_N // _R,),
        in_specs=[
            pl.BlockSpec((_R, _D), lambda i: (i, 0)),
            pl.BlockSpec((_D, _D), lambda i: (0, 0)),
            pl.BlockSpec((2, _R, _D), lambda i: (0, i, 0)),
        ],
        out_specs=[
            pl.BlockSpec((_R, _D), lambda i: (i, 0)),
            pl.BlockSpec((_R, 1), lambda i: (i, 0)),
        ],
        out_shape=[
            jax.ShapeDtypeStruct((_N, _D), _F32),
            jax.ShapeDtypeStruct((_N, 1), _F32),
        ],
    )(x, w, degp)


def _node_features(sp_ref, t_ref, dinv_ref, b_ref):
    s = sp_ref[0] + sp_ref[1] + t_ref[...]
    return jnp.maximum(dinv_ref[...] * s + b_ref[...], 0.0)


def _mid_body(sp_ref, t_ref, dinv_ref, b_ref, w_ref, tn_ref):
    x = _node_features(sp_ref, t_ref, dinv_ref, b_ref)
    tn_ref[...] = jnp.dot(x, w_ref[...],
                          preferred_element_type=_F32) * dinv_ref[...]


def _tc_mid(sp, t, dinv, b, w):
    return pl.pallas_call(
        _mid_body,
        grid=(_N // _R,),
        in_specs=[
            pl.BlockSpec((2, _R, _D), lambda i: (0, i, 0)),
            pl.BlockSpec((_R, _D), lambda i: (i, 0)),
            pl.BlockSpec((_R, 1), lambda i: (i, 0)),
            pl.BlockSpec((1, _D), lambda i: (0, 0)),
            pl.BlockSpec((_D, _D), lambda i: (0, 0)),
        ],
        out_specs=pl.BlockSpec((_R, _D), lambda i: (i, 0)),
        out_shape=jax.ShapeDtypeStruct((_N, _D), _F32),
    )(sp, t, dinv, b, w)


def _pool_body(sp_ref, t_ref, dinv_ref, b_ref, batch_ref, sums_ref, cnt_ref):
    i = pl.program_id(0)
    x = _node_features(sp_ref, t_ref, dinv_ref, b_ref)
    seg = lax.broadcasted_iota(jnp.int32, (_R, _G), 1)
    p = (batch_ref[...] == seg).astype(_F32)
    sums = lax.dot_general(p, x, (((0,), (0,)), ((), ())),
                           preferred_element_type=_F32)
    cnt = lax.dot_general(p, jnp.ones((_R, 1), _F32),
                          (((0,), (0,)), ((), ())),
                          preferred_element_type=_F32)

    @pl.when(i == 0)
    def _():
        sums_ref[...] = jnp.zeros_like(sums_ref)
        cnt_ref[...] = jnp.zeros_like(cnt_ref)

    sums_ref[...] += sums
    cnt_ref[...] += cnt


def _tc_pool(sp, t, dinv, b, batch):
    return pl.pallas_call(
        _pool_body,
        grid=(_N // _R,),
        in_specs=[
            pl.BlockSpec((2, _R, _D), lambda i: (0, i, 0)),
            pl.BlockSpec((_R, _D), lambda i: (i, 0)),
            pl.BlockSpec((_R, 1), lambda i: (i, 0)),
            pl.BlockSpec((1, _D), lambda i: (0, 0)),
            pl.BlockSpec((_R, 1), lambda i: (i, 0)),
        ],
        out_specs=[
            pl.BlockSpec((_G, _D), lambda i: (0, 0)),
            pl.BlockSpec((_G, 1), lambda i: (0, 0)),
        ],
        out_shape=[
            jax.ShapeDtypeStruct((_G, _D), _F32),
            jax.ShapeDtypeStruct((_G, 1), _F32),
        ],
    )(sp, t, dinv, b, batch)


def _mlp_body(s1_ref, c1_ref, s2_ref, c2_ref,
              w1_ref, b1_ref, w2_ref, b2_ref, w3_ref, b3_ref,
              w4_ref, b4_ref, w5_ref, b5_ref, out_ref):
    h1 = s1_ref[...] / jnp.maximum(c1_ref[...], 1.0)
    h2 = s2_ref[...] / jnp.maximum(c2_ref[...], 1.0)
    h = jnp.concatenate([h1, h2], axis=1)
    ws = [w1_ref, w2_ref, w3_ref, w4_ref, w5_ref]
    bs = [b1_ref, b2_ref, b3_ref, b4_ref, b5_ref]
    for li in range(5):
        h = jnp.dot(h, ws[li][...], preferred_element_type=_F32) + bs[li][...]
        if li < 4:
            h = jnp.maximum(h, 0.0)
    out_ref[...] = 1.0 / (1.0 + jnp.exp(-h))


def _tc_mlp(s1, c1, s2, c2, cls_params):
    flat = []
    for (w, b) in cls_params:
        flat.extend([w, b.reshape(1, -1)])
    return pl.pallas_call(
        _mlp_body,
        out_shape=jax.ShapeDtypeStruct((_G, 1), _F32),
    )(s1, c1, s2, c2, *flat)


def _encode(x, edge_index, batch, enc_params, zeros_row, ones_t):
    src3 = edge_index[0].astype(jnp.int32).reshape(_NW * _NG, _GS, _CH)
    dst3 = edge_index[1].astype(jnp.int32).reshape(_NW * _NG, _GS, _CH)
    degp = _sc_scatter_kernel()(ones_t, src3, dst3, zeros_row).reshape(
        _NC, _NP, _D)
    t, dinv = _tc_first(x, enc_params[0][0], degp)
    for li in range(1, 5):
        sp = _sc_scatter_kernel()(t, src3, dst3, zeros_row).reshape(
            _NC, _NP, _D)
        t = _tc_mid(sp, t, dinv, enc_params[li - 1][1].reshape(1, _D),
                    enc_params[li][0])
    sp = _sc_scatter_kernel()(t, src3, dst3, zeros_row).reshape(_NC, _NP, _D)
    batch2 = batch.astype(jnp.int32).reshape(_N, 1)
    return _tc_pool(sp, t, dinv, enc_params[4][1].reshape(1, _D), batch2)


def kernel(x_1, edge_index_1, x_1_batch, x_2, edge_index_2, x_2_batch,
           enc_W1, enc_b1, enc_W2, enc_b2, enc_W3, enc_b3, enc_W4, enc_b4,
           enc_W5, enc_b5, cls_W1, cls_b1, cls_W2, cls_b2, cls_W3, cls_b3,
           cls_W4, cls_b4, cls_W5, cls_b5):
    enc = [(enc_W1, enc_b1), (enc_W2, enc_b2), (enc_W3, enc_b3),
           (enc_W4, enc_b4), (enc_W5, enc_b5)]
    cls = [(cls_W1, cls_b1), (cls_W2, cls_b2), (cls_W3, cls_b3),
           (cls_W4, cls_b4), (cls_W5, cls_b5)]
    zeros_row = jnp.zeros((_WB, _D), _F32)
    ones_t = jnp.ones((_N, _D), _F32)
    s1, c1 = _encode(x_1, edge_index_1, x_1_batch, enc, zeros_row, ones_t)
    s2, c2 = _encode(x_2, edge_index_2, x_2_batch, enc, zeros_row, ones_t)
    return _tc_mlp(s1, c1, s2, c2, cls)


# speedup vs baseline: 19.7877x; 1.1476x over previous
import functools

import jax
import jax.numpy as jnp
from jax import lax
from jax.experimental import pallas as pl
from jax.experimental.pallas import tpu as pltpu
from jax.experimental.pallas import tpu_sc as plsc

_N = 10000
_E = 320000
_D = 128
_G = 32
_NC = 2
_NS = 16
_NW = _NC * _NS
_EPW = _E // _NW
_CH = 125
_GS = 4
_NG = 20
_NSTEP = _GS * _NG
_NP = 10240
_RPT = _NP // _NS
_WB = 64
_R = 2000
_F32 = jnp.float32


@functools.cache
def _sc_scatter_kernel():
    mesh = plsc.VectorSubcoreMesh(core_axis_name="c", subcore_axis_name="s")
    return functools.partial(
        pl.kernel, mesh=mesh,
        out_type=jax.ShapeDtypeStruct((_NC * _NP, _D), _F32),
        scratch_types=[
            pltpu.VMEM((_GS, _CH), jnp.int32),
            pltpu.VMEM((_GS, _CH), jnp.int32),
            pltpu.VMEM((_GS, _CH), jnp.int32),
            pltpu.VMEM((_GS, _CH), jnp.int32),
            pltpu.VMEM((_CH, _D), _F32),
            pltpu.VMEM((_CH, _D), _F32),
            pltpu.VMEM_SHARED((_NP, _D), _F32),
            pltpu.SemaphoreType.DMA,
            pltpu.SemaphoreType.DMA,
            pltpu.SemaphoreType.DMA,
            pltpu.SemaphoreType.DMA,
            pltpu.SemaphoreType.DMA,
            pltpu.SemaphoreType.DMA,
        ])(_sc_scatter_body)


def _sc_scatter_body(t_hbm, src_hbm, dst_hbm, zeros_hbm, out_hbm,
                     srcg0, dstg0, srcg1, dstg1, rows0, rows1, acc,
                     semi0, semi1, semg0, semg1, sems0, sems1):
    c = lax.axis_index("c")
    s = lax.axis_index("s")
    wid = c * _NS + s
    row0 = s * _RPT
    gbase = wid * _NG
    srcg, dstg = (srcg0, srcg1), (dstg0, dstg1)
    rows, semi, semg = (rows0, rows1), (semi0, semi1), (semg0, semg1)
    sems = (sems0, sems1)

    zbuf = rows0.at[pl.ds(0, _WB)]
    pltpu.sync_copy(zeros_hbm, zbuf)
    for k in range(_RPT // _WB):
        pltpu.sync_copy(zbuf, acc.at[pl.ds(row0 + k * _WB, _WB)])
    plsc.subcore_barrier()

    def stage(g_dyn, r):
        pltpu.async_copy(src_hbm.at[gbase + g_dyn], srcg[r], semi[r])
        pltpu.async_copy(dst_hbm.at[gbase + g_dyn], dstg[r], semi[r])

    def stage_wait(r):
        pltpu.make_async_copy(src_hbm.at[gbase], srcg[r], semi[r]).wait()
        pltpu.make_async_copy(dst_hbm.at[gbase], dstg[r], semi[r]).wait()

    def g_start(b):
        rg = (b // _GS) % 2
        pltpu.async_copy(t_hbm.at[srcg[rg].at[b % _GS]], rows[b % 2],
                         semg[b % 2])

    def g_wait(b):
        rg = (b // _GS) % 2
        pltpu.make_async_copy(t_hbm.at[srcg[rg].at[b % _GS]], rows[b % 2],
                              semg[b % 2]).wait()

    def s_args(b):
        rg = (b // _GS) % 2
        return rows[b % 2], acc.at[dstg[rg].at[b % _GS]], sems[b % 2]

    def emit(b, last=False):
        boundary = (b + 1) % _GS == 0
        prev = b - 1
        if prev >= 0 and (prev + 1) % _GS != 0:
            r, d, sm = s_args(prev)
            pltpu.make_async_copy(r, d, sm).wait()
        nxt = b + 1
        if not (last and b == 2 * _GS - 1):
            if nxt % _GS == 0:
                stage_wait((nxt // _GS) % 2)
            g_start(nxt)
        g_wait(b)
        r, d, sm = s_args(b)
        if boundary:
            pltpu.sync_copy(r, d, add=True)
        else:
            pltpu.async_copy(r, d, sm, add=True)

    stage(0, 0)
    stage_wait(0)
    g_start(0)
    stage(1, 1)

    def body(i, carry):
        for b in range(_GS):
            emit(b)
        stage(2 * i + 2, 0)
        for b in range(_GS, 2 * _GS):
            emit(b)
        stage(2 * i + 3, 1)
        return carry

    lax.fori_loop(0, _NG // 2 - 1, body, 0)
    for b in range(2 * _GS):
        emit(b, last=True)

    plsc.subcore_barrier()
    for k in range(_RPT // _WB):
        pltpu.sync_copy(acc.at[pl.ds(row0 + k * _WB, _WB)], zbuf)
        pltpu.sync_copy(zbuf, out_hbm.at[pl.ds(c * _NP + row0 + k * _WB,
                                               _WB)])


_EDV = _EPW // 16


@functools.cache
def _sc_degree_kernel():
    mesh = plsc.VectorSubcoreMesh(core_axis_name="c", subcore_axis_name="s")
    return functools.partial(
        pl.kernel, mesh=mesh,
        out_type=jax.ShapeDtypeStruct((_NW * _N,), _F32),
        compiler_params=pltpu.CompilerParams(needs_layout_passes=False),
        scratch_types=[
            pltpu.VMEM((1, _EPW), jnp.int32),
            pltpu.VMEM((_N,), _F32),
        ])(_sc_degree_body)


def _sc_degree_body(dst_hbm, zeros_hbm, out_hbm, dstb, degb):
    c = lax.axis_index("c")
    s = lax.axis_index("s")
    wid = c * _NS + s
    pltpu.sync_copy(zeros_hbm, degb)
    pltpu.sync_copy(dst_hbm.at[wid], dstb)
    ones16 = jnp.full((16,), 1.0, _F32)
    zero16 = jnp.zeros((16,), jnp.int32)
    lane = lax.iota(jnp.int32, 16)

    def body(i, carry):
        idx = plsc.load_gather(dstb, [zero16, i * 16 + lane])
        plsc.addupdate_scatter(degb, [idx], ones16)
        return carry

    lax.fori_loop(0, _EDV, body, 0)
    pltpu.sync_copy(degb, out_hbm.at[pl.ds(wid * _N, _N)])


def _first_body(x_ref, w_ref, degp_ref, t_ref, dinv_ref):
    deg = jnp.sum(degp_ref[...], axis=1, keepdims=True) + 1.0
    dinv = lax.rsqrt(deg)
    t_ref[...] = jnp.dot(x_ref[...], w_ref[...],
                         preferred_element_type=_F32) * dinv
    dinv_ref[...] = dinv


def _tc_first(x, w, degp):
    return pl.pallas_call(
        _first_body,
        grid=(_N // _R,),
        in_specs=[
            pl.BlockSpec((_R, _D), lambda i: (i, 0)),
            pl.BlockSpec((_D, _D), lambda i: (0, 0)),
            pl.BlockSpec((_R, _NW), lambda i: (i, 0)),
        ],
        out_specs=[
            pl.BlockSpec((_R, _D), lambda i: (i, 0)),
            pl.BlockSpec((_R, 1), lambda i: (i, 0)),
        ],
        out_shape=[
            jax.ShapeDtypeStruct((_N, _D), _F32),
            jax.ShapeDtypeStruct((_N, 1), _F32),
        ],
    )(x, w, degp)


def _node_features(sp_ref, t_ref, dinv_ref, b_ref):
    s = sp_ref[0] + sp_ref[1] + t_ref[...]
    return jnp.maximum(dinv_ref[...] * s + b_ref[...], 0.0)


def _mid_body(sp_ref, t_ref, dinv_ref, b_ref, w_ref, tn_ref):
    x = _node_features(sp_ref, t_ref, dinv_ref, b_ref)
    tn_ref[...] = jnp.dot(x, w_ref[...],
                          preferred_element_type=_F32) * dinv_ref[...]


def _tc_mid(sp, t, dinv, b, w):
    return pl.pallas_call(
        _mid_body,
        grid=(_N // _R,),
        in_specs=[
            pl.BlockSpec((2, _R, _D), lambda i: (0, i, 0)),
            pl.BlockSpec((_R, _D), lambda i: (i, 0)),
            pl.BlockSpec((_R, 1), lambda i: (i, 0)),
            pl.BlockSpec((1, _D), lambda i: (0, 0)),
            pl.BlockSpec((_D, _D), lambda i: (0, 0)),
        ],
        out_specs=pl.BlockSpec((_R, _D), lambda i: (i, 0)),
        out_shape=jax.ShapeDtypeStruct((_N, _D), _F32),
    )(sp, t, dinv, b, w)


def _pool_body(sp_ref, t_ref, dinv_ref, b_ref, batch_ref, sums_ref, cnt_ref):
    i = pl.program_id(0)
    x = _node_features(sp_ref, t_ref, dinv_ref, b_ref)
    seg = lax.broadcasted_iota(jnp.int32, (_R, _G), 1)
    p = (batch_ref[...] == seg).astype(_F32)
    sums = lax.dot_general(p, x, (((0,), (0,)), ((), ())),
                           preferred_element_type=_F32)
    cnt = lax.dot_general(p, jnp.ones((_R, 1), _F32),
                          (((0,), (0,)), ((), ())),
                          preferred_element_type=_F32)

    @pl.when(i == 0)
    def _():
        sums_ref[...] = jnp.zeros_like(sums_ref)
        cnt_ref[...] = jnp.zeros_like(cnt_ref)

    sums_ref[...] += sums
    cnt_ref[...] += cnt


def _tc_pool(sp, t, dinv, b, batch):
    return pl.pallas_call(
        _pool_body,
        grid=(_N // _R,),
        in_specs=[
            pl.BlockSpec((2, _R, _D), lambda i: (0, i, 0)),
            pl.BlockSpec((_R, _D), lambda i: (i, 0)),
            pl.BlockSpec((_R, 1), lambda i: (i, 0)),
            pl.BlockSpec((1, _D), lambda i: (0, 0)),
            pl.BlockSpec((_R, 1), lambda i: (i, 0)),
        ],
        out_specs=[
            pl.BlockSpec((_G, _D), lambda i: (0, 0)),
            pl.BlockSpec((_G, 1), lambda i: (0, 0)),
        ],
        out_shape=[
            jax.ShapeDtypeStruct((_G, _D), _F32),
            jax.ShapeDtypeStruct((_G, 1), _F32),
        ],
    )(sp, t, dinv, b, batch)


def _mlp_body(s1_ref, c1_ref, s2_ref, c2_ref,
              w1_ref, b1_ref, w2_ref, b2_ref, w3_ref, b3_ref,
              w4_ref, b4_ref, w5_ref, b5_ref, out_ref):
    h1 = s1_ref[...] / jnp.maximum(c1_ref[...], 1.0)
    h2 = s2_ref[...] / jnp.maximum(c2_ref[...], 1.0)
    h = jnp.concatenate([h1, h2], axis=1)
    ws = [w1_ref, w2_ref, w3_ref, w4_ref, w5_ref]
    bs = [b1_ref, b2_ref, b3_ref, b4_ref, b5_ref]
    for li in range(5):
        h = jnp.dot(h, ws[li][...], preferred_element_type=_F32) + bs[li][...]
        if li < 4:
            h = jnp.maximum(h, 0.0)
    out_ref[...] = 1.0 / (1.0 + jnp.exp(-h))


def _tc_mlp(s1, c1, s2, c2, cls_params):
    flat = []
    for (w, b) in cls_params:
        flat.extend([w, b.reshape(1, -1)])
    return pl.pallas_call(
        _mlp_body,
        out_shape=jax.ShapeDtypeStruct((_G, 1), _F32),
    )(s1, c1, s2, c2, *flat)


def _encode(x, edge_index, batch, enc_params, zeros_row, zeros_degn):
    src = edge_index[0].astype(jnp.int32)
    dst = edge_index[1].astype(jnp.int32)
    src3 = src.reshape(_NW * _NG, _GS, _CH)
    dst3 = dst.reshape(_NW * _NG, _GS, _CH)
    dst_d = dst.reshape(_NW, 1, _EPW)
    degp = _sc_degree_kernel()(dst_d, zeros_degn).reshape(_NW, _N).T
    t, dinv = _tc_first(x, enc_params[0][0], degp)
    for li in range(1, 5):
        sp = _sc_scatter_kernel()(t, src3, dst3, zeros_row).reshape(
            _NC, _NP, _D)
        t = _tc_mid(sp, t, dinv, enc_params[li - 1][1].reshape(1, _D),
                    enc_params[li][0])
    sp = _sc_scatter_kernel()(t, src3, dst3, zeros_row).reshape(_NC, _NP, _D)
    batch2 = batch.astype(jnp.int32).reshape(_N, 1)
    return _tc_pool(sp, t, dinv, enc_params[4][1].reshape(1, _D), batch2)


def kernel(x_1, edge_index_1, x_1_batch, x_2, edge_index_2, x_2_batch,
           enc_W1, enc_b1, enc_W2, enc_b2, enc_W3, enc_b3, enc_W4, enc_b4,
           enc_W5, enc_b5, cls_W1, cls_b1, cls_W2, cls_b2, cls_W3, cls_b3,
           cls_W4, cls_b4, cls_W5, cls_b5):
    enc = [(enc_W1, enc_b1), (enc_W2, enc_b2), (enc_W3, enc_b3),
           (enc_W4, enc_b4), (enc_W5, enc_b5)]
    cls = [(cls_W1, cls_b1), (cls_W2, cls_b2), (cls_W3, cls_b3),
           (cls_W4, cls_b4), (cls_W5, cls_b5)]
    zeros_row = jnp.zeros((_WB, _D), _F32)
    zeros_degn = jnp.zeros((_N,), _F32)
    s1, c1 = _encode(x_1, edge_index_1, x_1_batch, enc, zeros_row, zeros_degn)
    s2, c2 = _encode(x_2, edge_index_2, x_2_batch, enc, zeros_row, zeros_degn)
    return _tc_mlp(s1, c1, s2, c2, cls)
